# baseline (speedup 1.0000x reference)
import jax
import jax.numpy as jnp
from jax import lax
from jax.experimental import pallas as pl

_NBITS = 96
_HIDDEN = 32
_CH = 128
_HW = 1024
_PB = 8


def _body(msgT_ref, evenT_ref, oddT_ref, lat_ref, out_ref):
    evenT = evenT_ref[...]
    oddT = oddT_ref[...]
    diffT = oddT - evenT
    baseT = jnp.sum(evenT, axis=1)
    msgT = msgT_ref[...].astype(jnp.float32)
    auxT = lax.dot_general(
        diffT, msgT, (((1,), (0,)), ((), ())),
        preferred_element_type=jnp.float32) + baseT[:, None]

    x = lat_ref[...]
    for p in range(_PB):
        out_ref[pl.ds(0, _CH), p, :] = x[:, p, :].T
    b = x.shape[0]
    out_ref[pl.ds(_CH, _HIDDEN), :, :] = jnp.broadcast_to(
        auxT[:, None, :], (_HIDDEN, _PB, b))


def kernel(latents, msg, msg_embeddings):
    batch, ch, s1, s2 = latents.shape
    hw = s1 * s2
    lat = latents.transpose(0, 2, 3, 1).reshape(batch, hw, ch)
    msgT = msg.T
    evenT = msg_embeddings[0::2].T
    oddT = msg_embeddings[1::2].T

    grid = (hw // _PB,)
    outT = pl.pallas_call(
        _body,
        grid=grid,
        in_specs=[
            pl.BlockSpec((_NBITS, batch), lambda p: (0, 0)),
            pl.BlockSpec((_HIDDEN, _NBITS), lambda p: (0, 0)),
            pl.BlockSpec((_HIDDEN, _NBITS), lambda p: (0, 0)),
            pl.BlockSpec((batch, _PB, ch), lambda p: (0, p, 0)),
        ],
        out_specs=pl.BlockSpec((ch + _HIDDEN, _PB, batch), lambda p: (0, p, 0)),
        out_shape=jax.ShapeDtypeStruct((ch + _HIDDEN, hw, batch), jnp.float32),
    )(msgT, evenT, oddT, lat)
    return outT.reshape(ch + _HIDDEN, s1, s2, batch).transpose(3, 0, 1, 2)


# speedup vs baseline: 1.1292x; 1.1292x over previous
import jax
import jax.numpy as jnp
from jax import lax
from jax.experimental import pallas as pl

_NBITS = 96
_HIDDEN = 32
_CH = 128
_HW = 1024
_PB = 32
_BB = 128


def _body(msgT_ref, evenT_ref, oddT_ref, lat_ref, out_ref):
    evenT = evenT_ref[...]
    oddT = oddT_ref[...]
    diffT = oddT - evenT
    baseT = jnp.sum(evenT, axis=1)
    msgT = msgT_ref[...].astype(jnp.float32)
    auxT = lax.dot_general(
        diffT, msgT, (((1,), (0,)), ((), ())),
        preferred_element_type=jnp.float32) + baseT[:, None]

    x = lat_ref[...]
    for p in range(_PB):
        out_ref[pl.ds(0, _CH), p, :] = x[:, p, :].T
    b = x.shape[0]
    out_ref[pl.ds(_CH, _HIDDEN), :, :] = jnp.broadcast_to(
        auxT[:, None, :], (_HIDDEN, _PB, b))


def kernel(latents, msg, msg_embeddings):
    batch, ch, s1, s2 = latents.shape
    hw = s1 * s2
    lat = latents.transpose(0, 2, 3, 1).reshape(batch, hw, ch)
    msgT = msg.T
    evenT = msg_embeddings[0::2].T
    oddT = msg_embeddings[1::2].T

    grid = (batch // _BB, hw // _PB)
    outT = pl.pallas_call(
        _body,
        grid=grid,
        in_specs=[
            pl.BlockSpec((_NBITS, _BB), lambda b, p: (0, b)),
            pl.BlockSpec((_HIDDEN, _NBITS), lambda b, p: (0, 0)),
            pl.BlockSpec((_HIDDEN, _NBITS), lambda b, p: (0, 0)),
            pl.BlockSpec((_BB, _PB, ch), lambda b, p: (b, p, 0)),
        ],
        out_specs=pl.BlockSpec((ch + _HIDDEN, _PB, _BB), lambda b, p: (0, p, b)),
        out_shape=jax.ShapeDtypeStruct((ch + _HIDDEN, hw, batch), jnp.float32),
    )(msgT, evenT, oddT, lat)
    return outT.reshape(ch + _HIDDEN, s1, s2, batch).transpose(3, 0, 1, 2)


# speedup vs baseline: 1.2256x; 1.0853x over previous
import jax
import jax.numpy as jnp
from jax import lax
from jax.experimental import pallas as pl

_NBITS = 96
_HIDDEN = 32
_CH = 128
_HW = 1024
_PB = 16


def _body(msgT_ref, evenT_ref, oddT_ref, lat_ref, out_ref):
    evenT = evenT_ref[...]
    oddT = oddT_ref[...]
    diffT = oddT - evenT
    baseT = jnp.sum(evenT, axis=1)
    msgT = msgT_ref[...].astype(jnp.float32)
    auxT = lax.dot_general(
        diffT, msgT, (((1,), (0,)), ((), ())),
        preferred_element_type=jnp.float32) + baseT[:, None]

    x = lat_ref[...]
    for p in range(_PB):
        out_ref[pl.ds(0, _CH), p, :] = x[:, p, :].T
    b = x.shape[0]
    out_ref[pl.ds(_CH, _HIDDEN), :, :] = jnp.broadcast_to(
        auxT[:, None, :], (_HIDDEN, _PB, b))


def kernel(latents, msg, msg_embeddings):
    batch, ch, s1, s2 = latents.shape
    hw = s1 * s2
    lat = latents.transpose(0, 2, 3, 1).reshape(batch, hw, ch)
    msgT = msg.T
    evenT = msg_embeddings[0::2].T
    oddT = msg_embeddings[1::2].T

    grid = (hw // _PB,)
    outT = pl.pallas_call(
        _body,
        grid=grid,
        in_specs=[
            pl.BlockSpec((_NBITS, batch), lambda p: (0, 0)),
            pl.BlockSpec((_HIDDEN, _NBITS), lambda p: (0, 0)),
            pl.BlockSpec((_HIDDEN, _NBITS), lambda p: (0, 0)),
            pl.BlockSpec((batch, _PB, ch), lambda p: (0, p, 0)),
        ],
        out_specs=pl.BlockSpec((ch + _HIDDEN, _PB, batch), lambda p: (0, p, 0)),
        out_shape=jax.ShapeDtypeStruct((ch + _HIDDEN, hw, batch), jnp.float32),
    )(msgT, evenT, oddT, lat)
    return outT.reshape(ch + _HIDDEN, s1, s2, batch).transpose(3, 0, 1, 2)
